# baseline (speedup 1.0000x reference)
import functools

import jax
import jax.numpy as jnp
from jax import lax
from jax.experimental import pallas as pl
from jax.experimental.pallas import tpu as pltpu
from jax.experimental.pallas import tpu_sc as plsc

N = 50000
H = 64
E = 800000
L = 16
NW = 32
NB = 64
BW = 784
NPAD = 50176
EPAD = 819200
TR = 200
NCHUNK = 16
CH = 1600
ERCAP = 886784
CE = 2048
GB = 128
BLK = 512
NBLK = NPAD // BLK

_SC_PARAMS = pltpu.CompilerParams(needs_layout_passes=False)
_SC_PARAMS_NT = pltpu.CompilerParams(needs_layout_passes=False,
                                     use_tc_tiling_on_sc=False)


def _offsets_from_counts(cbuf, totb, stp, lane):
    for g in range(4):
        acc = jnp.zeros((L,), jnp.int32)
        for t in range(NW):
            acc = acc + cbuf[pl.ds(t * NB + g * 16, 16)]
        totb[pl.ds(g * 16, 16)] = acc
    run = jnp.int32(0)
    for g in range(4):
        v = totb[pl.ds(g * 16, 16)]
        va = (v + 1023) & jnp.int32(-1024)
        cs = plsc.cumsum(va)
        stp[pl.ds(g * 16, 16)] = cs - va + run
        run = run + cs[15]


def _k1_body(dst_hbm, w_hbm, cnt_hbm, cntl_hbm, degp_hbm,
             dstbuf, wbuf, cnt16, crow, zbuf, sh_deg, sem):
    cid = lax.axis_index("c")
    sid = lax.axis_index("s")
    wid = sid * 2 + cid
    lane = lax.iota(jnp.int32, 16)
    ones = jnp.ones((L,), jnp.int32)
    zf = jnp.zeros((L,), jnp.float32)
    zi = jnp.zeros((L,), jnp.int32)

    for g in range(3136 // 16):
        zbuf[pl.ds(g * 16, 16)] = zf
    pltpu.sync_copy(zbuf, sh_deg.at[pl.ds(sid * 3136, 3136)])
    for g in range(NB * 16 // 16):
        cnt16[pl.ds(g * 16, 16)] = zi
    plsc.subcore_barrier()

    def chunk(ci, carry):
        e0 = pl.multiple_of(wid * (TR * 128) + ci * CH, 8)
        pltpu.sync_copy(dst_hbm.at[pl.ds(e0, CH)], dstbuf)
        pltpu.sync_copy(w_hbm.at[pl.ds(e0, CH)], wbuf)
        def sub(sc, c0):
            base = sc * 800
            for gg in range(50):
                d = dstbuf[pl.ds(base + gg * 16, 16)]
                b = d // BW
                plsc.addupdate_scatter(cnt16, [b * 16 + lane], ones)
            return c0

        lax.fori_loop(0, 2, sub, jnp.int32(0))
        pltpu.sync_copy(wbuf, sh_deg.at[dstbuf], add=True)
        return carry

    lax.fori_loop(0, NCHUNK, chunk, jnp.int32(0))

    for g in range(4):
        acc = jnp.zeros((L,), jnp.int32)
        for l in range(16):
            acc = acc + plsc.load_gather(cnt16, [(g * 16 + lane) * 16 + l])
        crow[pl.ds(g * 16, 16)] = acc
    pltpu.sync_copy(crow, cnt_hbm.at[pl.ds(wid * NB, NB)])
    pltpu.sync_copy(cnt16, cntl_hbm.at[pl.ds(wid * NB * 16, NB * 16)])

    plsc.subcore_barrier()
    pltpu.sync_copy(sh_deg.at[pl.ds(sid * 3136, 3136)], zbuf)
    pltpu.sync_copy(zbuf, degp_hbm.at[pl.ds(cid * NPAD + sid * 3136, 3136)])


def _k1(dst2, w2):
    mesh = plsc.VectorSubcoreMesh(core_axis_name="c", subcore_axis_name="s")
    kfn = pl.kernel(
        _k1_body,
        out_type=(
            jax.ShapeDtypeStruct((NW * NB,), jnp.int32),
            jax.ShapeDtypeStruct((NW * NB * 16,), jnp.int32),
            jax.ShapeDtypeStruct((2 * NPAD,), jnp.float32),
        ),
        mesh=mesh,
        compiler_params=_SC_PARAMS,
        scratch_types=(
            pltpu.VMEM((CH,), jnp.int32),
            pltpu.VMEM((CH,), jnp.float32),
            pltpu.VMEM((NB * 16,), jnp.int32),
            pltpu.VMEM((NB,), jnp.int32),
            pltpu.VMEM((3136,), jnp.float32),
            pltpu.VMEM_SHARED((NPAD,), jnp.float32),
            pltpu.SemaphoreType.DMA,
        ),
    )
    return kfn(dst2, w2)


def _k2_body(src_hbm, dst_hbm, w_hbm, dinv_hbm, x_hbm, cnt_hbm, cntl_hbm,
             srcR, dlR, nrmR, sp_hbm,
             dtbl, xtbl, cbuf, totb, stp, goff, cur, ownl,
             srcb, dstb, wb, dlb, nrmb, ctb, posb, shif, zbuf, sh_s, sem):
    cid = lax.axis_index("c")
    sid = lax.axis_index("s")
    wid = sid * 2 + cid
    lane = lax.iota(jnp.int32, 16)
    zf = jnp.zeros((L,), jnp.float32)
    m1 = jnp.full((L,), -1, jnp.int32)

    for g in range(3136 // 16):
        zbuf[pl.ds(g * 16, 16)] = zf
    pltpu.sync_copy(zbuf, sh_s.at[pl.ds(sid * 3136, 3136)])
    plsc.subcore_barrier()

    pltpu.sync_copy(dinv_hbm, dtbl)
    pltpu.sync_copy(x_hbm, xtbl)
    pltpu.sync_copy(cnt_hbm, cbuf)
    _offsets_from_counts(cbuf, totb, stp, lane)

    for g in range(4):
        acc = jnp.zeros((L,), jnp.int32)
        for t in range(NW):
            v = cbuf[pl.ds(t * NB + g * 16, 16)]
            acc = acc + jnp.where(jnp.int32(t) < wid, v, jnp.int32(0))
        goff[pl.ds(g * 16, 16)] = acc + stp[pl.ds(g * 16, 16)]

    e0l = pl.multiple_of(wid * (NB * 16), 8)
    pltpu.sync_copy(cntl_hbm.at[pl.ds(e0l, NB * 16)], ownl)
    for b in range(NB):
        own16 = ownl[pl.ds(b * 16, 16)]
        excl = plsc.cumsum(own16) - own16
        base = plsc.load_gather(goff, [jnp.zeros((L,), jnp.int32) + b])
        cur[pl.ds(b * 16, 16)] = base + excl

    def chunk(ci, carry):
        e0 = pl.multiple_of(wid * (TR * 128) + ci * CH, 8)
        pltpu.sync_copy(src_hbm.at[pl.ds(e0, CH)], srcb)
        pltpu.sync_copy(dst_hbm.at[pl.ds(e0, CH)], dstb)
        pltpu.sync_copy(w_hbm.at[pl.ds(e0, CH)], wb)
        def sub(sc, c0):
          base = sc * 800
          for gg in range(50):
            j0 = base + gg * 16
            s16 = srcb[pl.ds(j0, 16)]
            d16 = dstb[pl.ds(j0, 16)]
            w16 = wb[pl.ds(j0, 16)]
            b16 = d16 // BW
            dlb[pl.ds(j0, 16)] = d16 - b16 * BW
            nv = plsc.load_gather(dtbl, [s16]) * w16 \
                * plsc.load_gather(dtbl, [d16])
            nrmb[pl.ds(j0, 16)] = nv
            ctb[pl.ds(j0, 16)] = nv * plsc.load_gather(xtbl, [s16])
            ci16 = b16 * 16 + lane
            pos = plsc.load_gather(cur, [ci16])
            posb[pl.ds(j0, 16)] = pos
            plsc.store_scatter(cur, [ci16], pos + 1)
          return c0

        lax.fori_loop(0, 2, sub, jnp.int32(0))
        descs = [
            pltpu.async_copy(srcb, srcR.at[posb], sem),
            pltpu.async_copy(dlb, dlR.at[posb], sem),
            pltpu.async_copy(nrmb, nrmR.at[posb], sem),
        ]
        pltpu.sync_copy(ctb, sh_s.at[dstb], add=True)
        for d in descs:
            d.wait()
        return carry

    lax.fori_loop(0, NCHUNK, chunk, jnp.int32(0))

    plsc.subcore_barrier()
    pltpu.sync_copy(sh_s.at[pl.ds(sid * 3136, 3136)], zbuf)
    pltpu.sync_copy(zbuf, sp_hbm.at[pl.ds(cid * NPAD + sid * 3136, 3136)])


def _k2(src2, dst2, w2, dinv, xpad, counts, cntl):
    mesh = plsc.VectorSubcoreMesh(core_axis_name="c", subcore_axis_name="s")
    kfn = pl.kernel(
        _k2_body,
        out_type=(
            jax.ShapeDtypeStruct((ERCAP,), jnp.int32),
            jax.ShapeDtypeStruct((ERCAP,), jnp.int32),
            jax.ShapeDtypeStruct((ERCAP,), jnp.float32),
            jax.ShapeDtypeStruct((2 * NPAD,), jnp.float32),
        ),
        mesh=mesh,
        compiler_params=_SC_PARAMS,
        scratch_types=(
            pltpu.VMEM((NPAD,), jnp.float32),
            pltpu.VMEM((NPAD,), jnp.float32),
            pltpu.VMEM((NW * NB,), jnp.int32),
            pltpu.VMEM((80,), jnp.int32),
            pltpu.VMEM((80,), jnp.int32),
            pltpu.VMEM((80,), jnp.int32),
            pltpu.VMEM((NB * 16,), jnp.int32),
            pltpu.VMEM((NB * 16,), jnp.int32),
            pltpu.VMEM((CH,), jnp.int32),
            pltpu.VMEM((CH,), jnp.int32),
            pltpu.VMEM((CH,), jnp.float32),
            pltpu.VMEM((CH,), jnp.int32),
            pltpu.VMEM((CH,), jnp.float32),
            pltpu.VMEM((CH,), jnp.float32),
            pltpu.VMEM((CH,), jnp.int32),
            pltpu.VMEM((48,), jnp.int32),
            pltpu.VMEM((3136,), jnp.float32),
            pltpu.VMEM_SHARED((NPAD,), jnp.float32),
            pltpu.SemaphoreType.DMA,
        ),
    )
    return kfn(src2, dst2, w2, dinv, xpad, counts, cntl)


def _agg_body(y_hbm, srcR, dlR, nrmR, cnt_hbm, zrows_hbm, agg_hbm,
              acc, cbuf, totb, stp, seb, dlb, nrb,
              idx0, idx1, rows0, rows1, sdl, snr, sem0, sem1):
    cid = lax.axis_index("c")
    sid = lax.axis_index("s")
    wid = sid * 2 + cid
    lane = lax.iota(jnp.int32, 16)

    pltpu.sync_copy(cnt_hbm, cbuf)
    _offsets_from_counts(cbuf, totb, stp, lane)

    idxb = (idx0, idx1)
    rows = (rows0, rows1)
    sems = (sem0, sem1)

    for mb in range(2):
        b = wid * 2 + mb
        pltpu.sync_copy(zrows_hbm, acc)
        bsplat = jnp.zeros((L,), jnp.int32) + b
        tot_b = plsc.load_gather(totb, [bsplat])[0]
        st_b = plsc.load_gather(stp, [bsplat])[0]
        nch = (tot_b + (CE - 1)) // CE

        def chunk(ci, carry):
            e0 = pl.multiple_of(st_b + ci * CE, 8)
            rem = tot_b - ci * CE
            pltpu.sync_copy(srcR.at[pl.ds(e0, CE)], seb)
            pltpu.sync_copy(dlR.at[pl.ds(e0, CE)], dlb)
            pltpu.sync_copy(nrmR.at[pl.ds(e0, CE)], nrb)
            descs = [None, None]

            def build_and_fire(k):
                def pgroup(g, c0):
                    j0 = k * GB + g * 16
                    sv = seb[pl.ds(j0, 16)]
                    sv = jnp.minimum(jnp.maximum(sv, jnp.int32(0)),
                                     jnp.int32(N - 1))
                    idxb[k % 2][pl.ds(g * 16, 16)] = sv
                    valid = (j0 + lane) < rem
                    nv = jnp.where(valid, nrb[pl.ds(j0, 16)],
                                   jnp.float32(0.0))
                    nrb[pl.ds(j0, 16)] = nv
                    dv = dlb[pl.ds(j0, 16)]
                    dv = jnp.minimum(jnp.maximum(dv, jnp.int32(0)),
                                     jnp.int32(BW - 1)) * H
                    dlb[pl.ds(j0, 16)] = dv
                    return c0

                lax.fori_loop(0, GB // 16, pgroup, jnp.int32(0))
                descs[k % 2] = pltpu.async_copy(
                    y_hbm.at[idxb[k % 2]], rows[k % 2], sems[k % 2])

            def compute(k):
                descs[k % 2].wait()
                r2d = rows[k % 2]

                def ebody(j4, c2):
                    j = j4 * 4
                    base = jnp.zeros((L,), jnp.int32) + (k * GB + j)
                    for u in range(4):
                        dv = plsc.load_gather(dlb, [base + u])
                        nb = plsc.load_gather(nrb, [base + u])
                        for c in range(4):
                            rv = r2d[j + u, pl.ds(c * 16, 16)]
                            plsc.addupdate_scatter(
                                acc, [dv + (c * 16 + lane)], rv * nb)
                    return c2

                lax.fori_loop(0, GB // 4, ebody, jnp.int32(0))

            build_and_fire(0)
            for k in range(1, CE // GB):
                build_and_fire(k)
                compute(k - 1)
            compute(CE // GB - 1)
            return carry

        lax.fori_loop(0, nch, chunk, jnp.int32(0))
        pltpu.sync_copy(acc, agg_hbm.at[pl.ds(b * (BW * H), BW * H)])


def _agg(y, srcR, dlR, nrmR, counts, zrows):
    mesh = plsc.VectorSubcoreMesh(core_axis_name="c", subcore_axis_name="s")
    kfn = pl.kernel(
        _agg_body,
        out_type=jax.ShapeDtypeStruct((NPAD * H,), jnp.float32),
        mesh=mesh,
        compiler_params=_SC_PARAMS_NT,
        scratch_types=(
            pltpu.VMEM((BW * H,), jnp.float32),
            pltpu.VMEM((NW * NB,), jnp.int32),
            pltpu.VMEM((80,), jnp.int32),
            pltpu.VMEM((80,), jnp.int32),
            pltpu.VMEM((CE,), jnp.int32),
            pltpu.VMEM((CE,), jnp.int32),
            pltpu.VMEM((CE,), jnp.float32),
            pltpu.VMEM((GB,), jnp.int32),
            pltpu.VMEM((GB,), jnp.int32),
            pltpu.VMEM((GB, H), jnp.float32),
            pltpu.VMEM((GB, H), jnp.float32),
            pltpu.SMEM((GB,), jnp.int32),
            pltpu.SMEM((GB,), jnp.float32),
            pltpu.SemaphoreType.DMA,
            pltpu.SemaphoreType.DMA,
        ),
    )
    return kfn(y, srcR, dlR, nrmR, counts, zrows)


def _tc1_body(s_ref, x_ref, di_ref, w0_ref, b0_ref, w1_ref, y_ref):
    di = di_ref[...]
    agg0 = s_ref[...] + di * di * x_ref[...]
    xx1 = jnp.maximum(agg0 * w0_ref[...] + b0_ref[...], 0.0)
    y_ref[...] = jnp.dot(xx1, w1_ref[...],
                         preferred_element_type=jnp.float32)


def _tc1(s, xpad, dinv, W0r, b0r, W1):
    col = pl.BlockSpec((BLK, 1), lambda i: (i, 0))
    full = lambda shape: pl.BlockSpec(shape, lambda i: (0, 0))
    return pl.pallas_call(
        _tc1_body,
        grid=(NBLK,),
        in_specs=[col, col, col, full((1, H)), full((1, H)), full((H, H))],
        out_specs=pl.BlockSpec((BLK, H), lambda i: (i, 0)),
        out_shape=jax.ShapeDtypeStruct((NPAD, H), jnp.float32),
    )(s, xpad, dinv, W0r, b0r, W1)


def _tc2_body(a_ref, y_ref, di_ref, b_ref, w_ref, o_ref):
    di = di_ref[...]
    xx = jnp.maximum(a_ref[...] + di * di * y_ref[...] + b_ref[...], 0.0)
    o_ref[...] = jnp.dot(xx, w_ref[...], preferred_element_type=jnp.float32)


def _tc2(agg1, y1, dinv, b1r, W2):
    col = pl.BlockSpec((BLK, 1), lambda i: (i, 0))
    mat = pl.BlockSpec((BLK, H), lambda i: (i, 0))
    full = lambda shape: pl.BlockSpec(shape, lambda i: (0, 0))
    return pl.pallas_call(
        _tc2_body,
        grid=(NBLK,),
        in_specs=[mat, mat, col, full((1, H)), full((H, H))],
        out_specs=mat,
        out_shape=jax.ShapeDtypeStruct((NPAD, H), jnp.float32),
    )(agg1, y1, dinv, b1r, W2)


def _tc3_body(a_ref, y_ref, di_ref, b2_ref, h_ref, wih_ref, whh_ref,
              bih_ref, bhh_ref, wfc_ref, bfc_ref, hn_ref, o_ref):
    di = di_ref[...]
    xx3 = jnp.maximum(a_ref[...] + di * di * y_ref[...] + b2_ref[...], 0.0)
    hh = h_ref[...]
    gi = jnp.dot(xx3, wih_ref[...],
                 preferred_element_type=jnp.float32) + bih_ref[...]
    gh = jnp.dot(hh, whh_ref[...],
                 preferred_element_type=jnp.float32) + bhh_ref[...]
    r = jax.nn.sigmoid(gi[:, 0:H] + gh[:, 0:H])
    z = jax.nn.sigmoid(gi[:, H:2 * H] + gh[:, H:2 * H])
    n = jnp.tanh(gi[:, 2 * H:3 * H] + r * gh[:, 2 * H:3 * H])
    hn = (1.0 - z) * n + z * hh
    hn_ref[...] = hn
    o_ref[...] = jnp.sum(hn * wfc_ref[...], axis=1,
                         keepdims=True) + bfc_ref[...]


def _tc3(agg2, y2, dinv, b2r, hpad, WihT, WhhT, bihr, bhhr, wfcr, bfcr):
    col = pl.BlockSpec((BLK, 1), lambda i: (i, 0))
    mat = pl.BlockSpec((BLK, H), lambda i: (i, 0))
    full = lambda shape: pl.BlockSpec(shape, lambda i: (0, 0))
    return pl.pallas_call(
        _tc3_body,
        grid=(NBLK,),
        in_specs=[mat, mat, col, full((1, H)), mat,
                  full((H, 3 * H)), full((H, 3 * H)),
                  full((1, 3 * H)), full((1, 3 * H)),
                  full((1, H)), full((1, 1))],
        out_specs=[mat, col],
        out_shape=[
            jax.ShapeDtypeStruct((NPAD, H), jnp.float32),
            jax.ShapeDtypeStruct((NPAD, 1), jnp.float32),
        ],
    )(agg2, y2, dinv, b2r, hpad, WihT, WhhT, bihr, bhhr, wfcr, bfcr)


def kernel(x, edge_index, edge_weight, h, W0, b0, W1, b1, W2, b2, Wih, Whh,
           bih, bhh, Wfc, bfc):
    f32 = jnp.float32
    xf = x.reshape(N).astype(f32)
    xpad = jnp.concatenate([xf, jnp.zeros((NPAD - N,), f32)])
    npad_e = EPAD - E
    src_p = jnp.concatenate(
        [edge_index[0], jnp.zeros((npad_e,), jnp.int32)])
    dst_p = jnp.concatenate(
        [edge_index[1], (jnp.arange(npad_e, dtype=jnp.int32) % N)])
    w_p = jnp.concatenate([edge_weight.astype(f32), jnp.zeros((npad_e,), f32)])
    src2 = src_p
    dst2 = dst_p
    w2 = w_p

    counts, cntl, degp = _k1(dst2, w2)
    deg = degp[:NPAD] + degp[NPAD:] + 1.0
    dinv = deg ** -0.5

    srcR, dlR, nrmR, sp = _k2(src2, dst2, w2, dinv, xpad, counts, cntl)
    s = sp[:NPAD] + sp[NPAD:]

    zrows = jnp.zeros((BW * H,), f32)
    s2 = s.reshape(NPAD, 1)
    x2 = xpad.reshape(NPAD, 1)
    di2 = dinv.reshape(NPAD, 1)
    W0r = W0.reshape(1, H)
    b0r = b0.reshape(1, H)
    y1 = _tc1(s2, x2, di2, W0r, b0r, W1)

    agg1 = _agg(y1, srcR, dlR, nrmR, counts, zrows).reshape(NPAD, H)
    y2 = _tc2(agg1, y1, di2, b1.reshape(1, H), W2)
    agg2 = _agg(y2, srcR, dlR, nrmR, counts, zrows).reshape(NPAD, H)

    hpad = jnp.concatenate(
        [h.reshape(N, H).astype(f32), jnp.zeros((NPAD - N, H), f32)])
    hn, out = _tc3(agg2, y2, di2, b2.reshape(1, H), hpad,
                   Wih.T, Whh.T, bih.reshape(1, 3 * H),
                   bhh.reshape(1, 3 * H), Wfc.reshape(1, H),
                   bfc.reshape(1, 1))
    return (out[:N].reshape(1, N, 1), hn[:N].reshape(1, N, H))


# speedup vs baseline: 1.0008x; 1.0008x over previous
import functools

import jax
import jax.numpy as jnp
from jax import lax
from jax.experimental import pallas as pl
from jax.experimental.pallas import tpu as pltpu
from jax.experimental.pallas import tpu_sc as plsc

N = 50000
H = 64
E = 800000
L = 16
NW = 32
NB = 64
BW = 784
NPAD = 50176
EPAD = 819200
TR = 200
NCHUNK = 16
CH = 1600
ERCAP = 886784
CE = 2048
GB = 128
BLK = 512
NBLK = NPAD // BLK

_SC_PARAMS = pltpu.CompilerParams(needs_layout_passes=False)
_SC_PARAMS_NT = pltpu.CompilerParams(needs_layout_passes=False,
                                     use_tc_tiling_on_sc=False)


def _offsets_from_counts(cbuf, totb, stp, lane):
    for g in range(4):
        acc = jnp.zeros((L,), jnp.int32)
        for t in range(NW):
            acc = acc + cbuf[pl.ds(t * NB + g * 16, 16)]
        totb[pl.ds(g * 16, 16)] = acc
    run = jnp.int32(0)
    for g in range(4):
        v = totb[pl.ds(g * 16, 16)]
        va = (v + 1023) & jnp.int32(-1024)
        cs = plsc.cumsum(va)
        stp[pl.ds(g * 16, 16)] = cs - va + run
        run = run + cs[15]


def _k1_body(dst_hbm, w_hbm, cnt_hbm, cntl_hbm, degp_hbm,
             dstbuf, wbuf, cnt16, crow, zbuf, sh_deg, sem):
    cid = lax.axis_index("c")
    sid = lax.axis_index("s")
    wid = sid * 2 + cid
    lane = lax.iota(jnp.int32, 16)
    ones = jnp.ones((L,), jnp.int32)
    zf = jnp.zeros((L,), jnp.float32)
    zi = jnp.zeros((L,), jnp.int32)

    for g in range(3136 // 16):
        zbuf[pl.ds(g * 16, 16)] = zf
    pltpu.sync_copy(zbuf, sh_deg.at[pl.ds(sid * 3136, 3136)])
    for g in range(NB * 16 // 16):
        cnt16[pl.ds(g * 16, 16)] = zi
    plsc.subcore_barrier()

    def chunk(ci, carry):
        e0 = pl.multiple_of(wid * (TR * 128) + ci * CH, 8)
        pltpu.sync_copy(dst_hbm.at[pl.ds(e0, CH)], dstbuf)
        pltpu.sync_copy(w_hbm.at[pl.ds(e0, CH)], wbuf)
        def sub(sc, c0):
            base = sc * 800
            for gg in range(50):
                d = dstbuf[pl.ds(base + gg * 16, 16)]
                b = d // BW
                plsc.addupdate_scatter(cnt16, [b * 16 + lane], ones)
            return c0

        lax.fori_loop(0, 2, sub, jnp.int32(0))
        pltpu.sync_copy(wbuf, sh_deg.at[dstbuf], add=True)
        return carry

    lax.fori_loop(0, NCHUNK, chunk, jnp.int32(0))

    for g in range(4):
        acc = jnp.zeros((L,), jnp.int32)
        for l in range(16):
            acc = acc + plsc.load_gather(cnt16, [(g * 16 + lane) * 16 + l])
        crow[pl.ds(g * 16, 16)] = acc
    pltpu.sync_copy(crow, cnt_hbm.at[pl.ds(wid * NB, NB)])
    pltpu.sync_copy(cnt16, cntl_hbm.at[pl.ds(wid * NB * 16, NB * 16)])

    plsc.subcore_barrier()
    pltpu.sync_copy(sh_deg.at[pl.ds(sid * 3136, 3136)], zbuf)
    pltpu.sync_copy(zbuf, degp_hbm.at[pl.ds(cid * NPAD + sid * 3136, 3136)])


def _k1(dst2, w2):
    mesh = plsc.VectorSubcoreMesh(core_axis_name="c", subcore_axis_name="s")
    kfn = pl.kernel(
        _k1_body,
        out_type=(
            jax.ShapeDtypeStruct((NW * NB,), jnp.int32),
            jax.ShapeDtypeStruct((NW * NB * 16,), jnp.int32),
            jax.ShapeDtypeStruct((2 * NPAD,), jnp.float32),
        ),
        mesh=mesh,
        compiler_params=_SC_PARAMS,
        scratch_types=(
            pltpu.VMEM((CH,), jnp.int32),
            pltpu.VMEM((CH,), jnp.float32),
            pltpu.VMEM((NB * 16,), jnp.int32),
            pltpu.VMEM((NB,), jnp.int32),
            pltpu.VMEM((3136,), jnp.float32),
            pltpu.VMEM_SHARED((NPAD,), jnp.float32),
            pltpu.SemaphoreType.DMA,
        ),
    )
    return kfn(dst2, w2)


def _k2_body(src_hbm, dst_hbm, w_hbm, dinv_hbm, x_hbm, cnt_hbm, cntl_hbm,
             srcR, dlR, nrmR, sp_hbm,
             dtbl, xtbl, cbuf, totb, stp, goff, cur, ownl,
             srcb, dstb, wb, dlb, nrmb, ctb, posb, shif, zbuf, sh_s, sem):
    cid = lax.axis_index("c")
    sid = lax.axis_index("s")
    wid = sid * 2 + cid
    lane = lax.iota(jnp.int32, 16)
    zf = jnp.zeros((L,), jnp.float32)
    m1 = jnp.full((L,), -1, jnp.int32)

    for g in range(3136 // 16):
        zbuf[pl.ds(g * 16, 16)] = zf
    pltpu.sync_copy(zbuf, sh_s.at[pl.ds(sid * 3136, 3136)])
    plsc.subcore_barrier()

    pltpu.sync_copy(dinv_hbm, dtbl)
    pltpu.sync_copy(x_hbm, xtbl)
    pltpu.sync_copy(cnt_hbm, cbuf)
    _offsets_from_counts(cbuf, totb, stp, lane)

    for g in range(4):
        acc = jnp.zeros((L,), jnp.int32)
        for t in range(NW):
            v = cbuf[pl.ds(t * NB + g * 16, 16)]
            acc = acc + jnp.where(jnp.int32(t) < wid, v, jnp.int32(0))
        goff[pl.ds(g * 16, 16)] = acc + stp[pl.ds(g * 16, 16)]

    e0l = pl.multiple_of(wid * (NB * 16), 8)
    pltpu.sync_copy(cntl_hbm.at[pl.ds(e0l, NB * 16)], ownl)
    for b in range(NB):
        own16 = ownl[pl.ds(b * 16, 16)]
        excl = plsc.cumsum(own16) - own16
        base = plsc.load_gather(goff, [jnp.zeros((L,), jnp.int32) + b])
        cur[pl.ds(b * 16, 16)] = base + excl

    def chunk(ci, carry):
        e0 = pl.multiple_of(wid * (TR * 128) + ci * CH, 8)
        pltpu.sync_copy(src_hbm.at[pl.ds(e0, CH)], srcb)
        pltpu.sync_copy(dst_hbm.at[pl.ds(e0, CH)], dstb)
        pltpu.sync_copy(w_hbm.at[pl.ds(e0, CH)], wb)
        def sub(sc, c0):
          base = sc * 800
          for gg in range(50):
            j0 = base + gg * 16
            s16 = srcb[pl.ds(j0, 16)]
            d16 = dstb[pl.ds(j0, 16)]
            w16 = wb[pl.ds(j0, 16)]
            b16 = d16 // BW
            dlb[pl.ds(j0, 16)] = d16 - b16 * BW
            nv = plsc.load_gather(dtbl, [s16]) * w16 \
                * plsc.load_gather(dtbl, [d16])
            nrmb[pl.ds(j0, 16)] = nv
            ctb[pl.ds(j0, 16)] = nv * plsc.load_gather(xtbl, [s16])
            ci16 = b16 * 16 + lane
            pos = plsc.load_gather(cur, [ci16])
            posb[pl.ds(j0, 16)] = pos
            plsc.store_scatter(cur, [ci16], pos + 1)
          return c0

        lax.fori_loop(0, 2, sub, jnp.int32(0))
        descs = [
            pltpu.async_copy(srcb, srcR.at[posb], sem),
            pltpu.async_copy(dlb, dlR.at[posb], sem),
            pltpu.async_copy(nrmb, nrmR.at[posb], sem),
        ]
        pltpu.sync_copy(ctb, sh_s.at[dstb], add=True)
        for d in descs:
            d.wait()
        return carry

    lax.fori_loop(0, NCHUNK, chunk, jnp.int32(0))

    plsc.subcore_barrier()
    pltpu.sync_copy(sh_s.at[pl.ds(sid * 3136, 3136)], zbuf)
    pltpu.sync_copy(zbuf, sp_hbm.at[pl.ds(cid * NPAD + sid * 3136, 3136)])


def _k2(src2, dst2, w2, dinv, xpad, counts, cntl):
    mesh = plsc.VectorSubcoreMesh(core_axis_name="c", subcore_axis_name="s")
    kfn = pl.kernel(
        _k2_body,
        out_type=(
            jax.ShapeDtypeStruct((ERCAP,), jnp.int32),
            jax.ShapeDtypeStruct((ERCAP,), jnp.int32),
            jax.ShapeDtypeStruct((ERCAP,), jnp.float32),
            jax.ShapeDtypeStruct((2 * NPAD,), jnp.float32),
        ),
        mesh=mesh,
        compiler_params=_SC_PARAMS,
        scratch_types=(
            pltpu.VMEM((NPAD,), jnp.float32),
            pltpu.VMEM((NPAD,), jnp.float32),
            pltpu.VMEM((NW * NB,), jnp.int32),
            pltpu.VMEM((80,), jnp.int32),
            pltpu.VMEM((80,), jnp.int32),
            pltpu.VMEM((80,), jnp.int32),
            pltpu.VMEM((NB * 16,), jnp.int32),
            pltpu.VMEM((NB * 16,), jnp.int32),
            pltpu.VMEM((CH,), jnp.int32),
            pltpu.VMEM((CH,), jnp.int32),
            pltpu.VMEM((CH,), jnp.float32),
            pltpu.VMEM((CH,), jnp.int32),
            pltpu.VMEM((CH,), jnp.float32),
            pltpu.VMEM((CH,), jnp.float32),
            pltpu.VMEM((CH,), jnp.int32),
            pltpu.VMEM((48,), jnp.int32),
            pltpu.VMEM((3136,), jnp.float32),
            pltpu.VMEM_SHARED((NPAD,), jnp.float32),
            pltpu.SemaphoreType.DMA,
        ),
    )
    return kfn(src2, dst2, w2, dinv, xpad, counts, cntl)


def _agg_body(y_hbm, srcR, dlR, nrmR, cnt_hbm, zrows_hbm, agg_hbm,
              acc, cbuf, totb, stp, seb, dlb, nrb,
              idx0, idx1, rows0, rows1, sdl, snr, sem0, sem1):
    cid = lax.axis_index("c")
    sid = lax.axis_index("s")
    wid = sid * 2 + cid
    lane = lax.iota(jnp.int32, 16)

    pltpu.sync_copy(cnt_hbm, cbuf)
    _offsets_from_counts(cbuf, totb, stp, lane)

    idxb = (idx0, idx1)
    rows = (rows0, rows1)
    sems = (sem0, sem1)

    for mb in range(2):
        b = wid * 2 + mb
        pltpu.sync_copy(zrows_hbm, acc)
        bsplat = jnp.zeros((L,), jnp.int32) + b
        tot_b = plsc.load_gather(totb, [bsplat])[0]
        st_b = plsc.load_gather(stp, [bsplat])[0]
        nch = (tot_b + (CE - 1)) // CE

        def chunk(ci, carry):
            e0 = pl.multiple_of(st_b + ci * CE, 8)
            rem = tot_b - ci * CE
            pltpu.sync_copy(srcR.at[pl.ds(e0, CE)], seb)
            pltpu.sync_copy(dlR.at[pl.ds(e0, CE)], dlb)
            pltpu.sync_copy(nrmR.at[pl.ds(e0, CE)], nrb)
            descs = [None, None]

            def build_and_fire(k):
                def pgroup(g, c0):
                    j0 = k * GB + g * 16
                    sv = seb[pl.ds(j0, 16)]
                    sv = jnp.minimum(jnp.maximum(sv, jnp.int32(0)),
                                     jnp.int32(N - 1))
                    idxb[k % 2][pl.ds(g * 16, 16)] = sv
                    valid = (j0 + lane) < rem
                    nv = jnp.where(valid, nrb[pl.ds(j0, 16)],
                                   jnp.float32(0.0))
                    nrb[pl.ds(j0, 16)] = nv
                    dv = dlb[pl.ds(j0, 16)]
                    dv = jnp.minimum(jnp.maximum(dv, jnp.int32(0)),
                                     jnp.int32(BW - 1)) * H
                    dlb[pl.ds(j0, 16)] = dv
                    return c0

                lax.fori_loop(0, GB // 16, pgroup, jnp.int32(0))
                descs[k % 2] = pltpu.async_copy(
                    y_hbm.at[idxb[k % 2]], rows[k % 2], sems[k % 2])

            def compute(k):
                descs[k % 2].wait()
                r2d = rows[k % 2]

                def ebody(j4, c2):
                    j = j4 * 8
                    base = jnp.zeros((L,), jnp.int32) + (k * GB + j)
                    for u in range(8):
                        dv = plsc.load_gather(dlb, [base + u])
                        nb = plsc.load_gather(nrb, [base + u])
                        for c in range(4):
                            rv = r2d[j + u, pl.ds(c * 16, 16)]
                            plsc.addupdate_scatter(
                                acc, [dv + (c * 16 + lane)], rv * nb)
                    return c2

                lax.fori_loop(0, GB // 8, ebody, jnp.int32(0))

            build_and_fire(0)
            for k in range(1, CE // GB):
                build_and_fire(k)
                compute(k - 1)
            compute(CE // GB - 1)
            return carry

        lax.fori_loop(0, nch, chunk, jnp.int32(0))
        pltpu.sync_copy(acc, agg_hbm.at[pl.ds(b * (BW * H), BW * H)])


def _agg(y, srcR, dlR, nrmR, counts, zrows):
    mesh = plsc.VectorSubcoreMesh(core_axis_name="c", subcore_axis_name="s")
    kfn = pl.kernel(
        _agg_body,
        out_type=jax.ShapeDtypeStruct((NPAD * H,), jnp.float32),
        mesh=mesh,
        compiler_params=_SC_PARAMS_NT,
        scratch_types=(
            pltpu.VMEM((BW * H,), jnp.float32),
            pltpu.VMEM((NW * NB,), jnp.int32),
            pltpu.VMEM((80,), jnp.int32),
            pltpu.VMEM((80,), jnp.int32),
            pltpu.VMEM((CE,), jnp.int32),
            pltpu.VMEM((CE,), jnp.int32),
            pltpu.VMEM((CE,), jnp.float32),
            pltpu.VMEM((GB,), jnp.int32),
            pltpu.VMEM((GB,), jnp.int32),
            pltpu.VMEM((GB, H), jnp.float32),
            pltpu.VMEM((GB, H), jnp.float32),
            pltpu.SMEM((GB,), jnp.int32),
            pltpu.SMEM((GB,), jnp.float32),
            pltpu.SemaphoreType.DMA,
            pltpu.SemaphoreType.DMA,
        ),
    )
    return kfn(y, srcR, dlR, nrmR, counts, zrows)


def _tc1_body(s_ref, x_ref, di_ref, w0_ref, b0_ref, w1_ref, y_ref):
    di = di_ref[...]
    agg0 = s_ref[...] + di * di * x_ref[...]
    xx1 = jnp.maximum(agg0 * w0_ref[...] + b0_ref[...], 0.0)
    y_ref[...] = jnp.dot(xx1, w1_ref[...],
                         preferred_element_type=jnp.float32)


def _tc1(s, xpad, dinv, W0r, b0r, W1):
    col = pl.BlockSpec((BLK, 1), lambda i: (i, 0))
    full = lambda shape: pl.BlockSpec(shape, lambda i: (0, 0))
    return pl.pallas_call(
        _tc1_body,
        grid=(NBLK,),
        in_specs=[col, col, col, full((1, H)), full((1, H)), full((H, H))],
        out_specs=pl.BlockSpec((BLK, H), lambda i: (i, 0)),
        out_shape=jax.ShapeDtypeStruct((NPAD, H), jnp.float32),
    )(s, xpad, dinv, W0r, b0r, W1)


def _tc2_body(a_ref, y_ref, di_ref, b_ref, w_ref, o_ref):
    di = di_ref[...]
    xx = jnp.maximum(a_ref[...] + di * di * y_ref[...] + b_ref[...], 0.0)
    o_ref[...] = jnp.dot(xx, w_ref[...], preferred_element_type=jnp.float32)


def _tc2(agg1, y1, dinv, b1r, W2):
    col = pl.BlockSpec((BLK, 1), lambda i: (i, 0))
    mat = pl.BlockSpec((BLK, H), lambda i: (i, 0))
    full = lambda shape: pl.BlockSpec(shape, lambda i: (0, 0))
    return pl.pallas_call(
        _tc2_body,
        grid=(NBLK,),
        in_specs=[mat, mat, col, full((1, H)), full((H, H))],
        out_specs=mat,
        out_shape=jax.ShapeDtypeStruct((NPAD, H), jnp.float32),
    )(agg1, y1, dinv, b1r, W2)


def _tc3_body(a_ref, y_ref, di_ref, b2_ref, h_ref, wih_ref, whh_ref,
              bih_ref, bhh_ref, wfc_ref, bfc_ref, hn_ref, o_ref):
    di = di_ref[...]
    xx3 = jnp.maximum(a_ref[...] + di * di * y_ref[...] + b2_ref[...], 0.0)
    hh = h_ref[...]
    gi = jnp.dot(xx3, wih_ref[...],
                 preferred_element_type=jnp.float32) + bih_ref[...]
    gh = jnp.dot(hh, whh_ref[...],
                 preferred_element_type=jnp.float32) + bhh_ref[...]
    r = jax.nn.sigmoid(gi[:, 0:H] + gh[:, 0:H])
    z = jax.nn.sigmoid(gi[:, H:2 * H] + gh[:, H:2 * H])
    n = jnp.tanh(gi[:, 2 * H:3 * H] + r * gh[:, 2 * H:3 * H])
    hn = (1.0 - z) * n + z * hh
    hn_ref[...] = hn
    o_ref[...] = jnp.sum(hn * wfc_ref[...], axis=1,
                         keepdims=True) + bfc_ref[...]


def _tc3(agg2, y2, dinv, b2r, hpad, WihT, WhhT, bihr, bhhr, wfcr, bfcr):
    col = pl.BlockSpec((BLK, 1), lambda i: (i, 0))
    mat = pl.BlockSpec((BLK, H), lambda i: (i, 0))
    full = lambda shape: pl.BlockSpec(shape, lambda i: (0, 0))
    return pl.pallas_call(
        _tc3_body,
        grid=(NBLK,),
        in_specs=[mat, mat, col, full((1, H)), mat,
                  full((H, 3 * H)), full((H, 3 * H)),
                  full((1, 3 * H)), full((1, 3 * H)),
                  full((1, H)), full((1, 1))],
        out_specs=[mat, col],
        out_shape=[
            jax.ShapeDtypeStruct((NPAD, H), jnp.float32),
            jax.ShapeDtypeStruct((NPAD, 1), jnp.float32),
        ],
    )(agg2, y2, dinv, b2r, hpad, WihT, WhhT, bihr, bhhr, wfcr, bfcr)


def kernel(x, edge_index, edge_weight, h, W0, b0, W1, b1, W2, b2, Wih, Whh,
           bih, bhh, Wfc, bfc):
    f32 = jnp.float32
    xf = x.reshape(N).astype(f32)
    xpad = jnp.concatenate([xf, jnp.zeros((NPAD - N,), f32)])
    npad_e = EPAD - E
    src_p = jnp.concatenate(
        [edge_index[0], jnp.zeros((npad_e,), jnp.int32)])
    dst_p = jnp.concatenate(
        [edge_index[1], (jnp.arange(npad_e, dtype=jnp.int32) % N)])
    w_p = jnp.concatenate([edge_weight.astype(f32), jnp.zeros((npad_e,), f32)])
    src2 = src_p
    dst2 = dst_p
    w2 = w_p

    counts, cntl, degp = _k1(dst2, w2)
    deg = degp[:NPAD] + degp[NPAD:] + 1.0
    dinv = deg ** -0.5

    srcR, dlR, nrmR, sp = _k2(src2, dst2, w2, dinv, xpad, counts, cntl)
    s = sp[:NPAD] + sp[NPAD:]

    zrows = jnp.zeros((BW * H,), f32)
    s2 = s.reshape(NPAD, 1)
    x2 = xpad.reshape(NPAD, 1)
    di2 = dinv.reshape(NPAD, 1)
    W0r = W0.reshape(1, H)
    b0r = b0.reshape(1, H)
    y1 = _tc1(s2, x2, di2, W0r, b0r, W1)

    agg1 = _agg(y1, srcR, dlR, nrmR, counts, zrows).reshape(NPAD, H)
    y2 = _tc2(agg1, y1, di2, b1.reshape(1, H), W2)
    agg2 = _agg(y2, srcR, dlR, nrmR, counts, zrows).reshape(NPAD, H)

    hpad = jnp.concatenate(
        [h.reshape(N, H).astype(f32), jnp.zeros((NPAD - N, H), f32)])
    hn, out = _tc3(agg2, y2, di2, b2.reshape(1, H), hpad,
                   Wih.T, Whh.T, bih.reshape(1, 3 * H),
                   bhh.reshape(1, 3 * H), Wfc.reshape(1, H),
                   bfc.reshape(1, 1))
    return (out[:N].reshape(1, N, 1), hn[:N].reshape(1, N, H))
